# Initial kernel scaffold; baseline (speedup 1.0000x reference)
import functools

import numpy as np
import jax
import jax.numpy as jnp
from jax import lax
from jax.experimental import pallas as pl
from jax.experimental.pallas import tpu as pltpu
from jax.experimental.pallas import tpu_sc as plsc

_L = 16
_T = 19
_TSIZE = 1 << _T
_F = 2
_B = 262144
_NC, _NS, _LANES = 2, 16, 16
_NW = _NC * _NS
_PTS = _B // _NW
_CHUNK = 1024
_NIDX = 4 * _CHUNK

_bfac = np.exp((np.log(512.0) - np.log(16.0)) / (_L - 1))
_RES = [int(v) for v in np.floor(16.0 * _bfac ** np.arange(_L)).astype(np.int64)]
_PRIME = -1640531535
_MASK = _TSIZE - 1

_mesh = plsc.VectorSubcoreMesh(core_axis_name="c", subcore_axis_name="s")


@functools.partial(
    pl.kernel,
    out_type=jax.ShapeDtypeStruct((_B, _L * _F), jnp.float32),
    mesh=_mesh,
    scratch_types=[
        pltpu.VMEM((2, _PTS), jnp.float32),
        pltpu.VMEM((_CHUNK, _L * _F), jnp.float32),
        pltpu.VMEM((2, _CHUNK), jnp.float32),
        pltpu.VMEM((_NIDX,), jnp.int32),
        pltpu.VMEM((_NIDX, _F), jnp.float32),
        pltpu.SemaphoreType.DMA,
    ],
)
def _ngp_sc(xt_hbm, tab_hbm, out_hbm, xv, ov, fr, ixv, rv, sem):
    wid = lax.axis_index("c") * _NS + lax.axis_index("s")
    base = wid * _PTS
    pltpu.sync_copy(xt_hbm.at[:, pl.ds(base, _PTS)], xv)

    @pl.loop(0, _PTS, step=_LANES)
    def _norm(p):
        sl = pl.ds(p, _LANES)
        xv[0, sl] = (xv[0, sl] + 90.0) / 180.0
        xv[1, sl] = xv[1, sl] / 360.0

    iot = lax.iota(jnp.int32, _LANES)
    zc = jnp.zeros((_LANES,), jnp.int32)
    oc = jnp.ones((_LANES,), jnp.int32)

    @pl.loop(0, _PTS, step=_CHUNK)
    def _chunk(co):
        for l in range(_L):
            rf = float(_RES[l])
            rowbase = l * _TSIZE

            @pl.loop(0, _CHUNK, step=_LANES)
            def _hash_pass(q):
                sl = pl.ds(q, _LANES)
                sx = xv[0, pl.ds(co + q, _LANES)] * rf
                sy = xv[1, pl.ds(co + q, _LANES)] * rf
                gx = sx.astype(jnp.int32)
                gy = sy.astype(jnp.int32)
                fr[0, sl] = sx - gx.astype(jnp.float32)
                fr[1, sl] = sy - gy.astype(jnp.float32)
                hy0 = gy * _PRIME
                hy1 = hy0 + _PRIME
                gx1 = gx + 1
                ixv[sl] = ((gx ^ hy0) & _MASK) + rowbase
                ixv[pl.ds(_CHUNK + q, _LANES)] = ((gx1 ^ hy0) & _MASK) + rowbase
                ixv[pl.ds(2 * _CHUNK + q, _LANES)] = ((gx ^ hy1) & _MASK) + rowbase
                ixv[pl.ds(3 * _CHUNK + q, _LANES)] = ((gx1 ^ hy1) & _MASK) + rowbase

            pltpu.async_copy(tab_hbm.at[ixv], rv, sem).wait()

            col0 = jnp.full((_LANES,), 2 * l, jnp.int32)
            col1 = jnp.full((_LANES,), 2 * l + 1, jnp.int32)

            @pl.loop(0, _CHUNK, step=_LANES)
            def _interp_pass(q):
                sl = pl.ds(q, _LANES)
                fx = fr[0, sl]
                fy = fr[1, sl]
                wx1 = fx
                wx0 = 1.0 - fx
                wy1 = fy
                wy0 = 1.0 - fy
                row = q + iot
                acc0 = None
                acc1 = None
                for c, w in enumerate((wx0 * wy0, wx1 * wy0, wx0 * wy1, wx1 * wy1)):
                    rr = row + c * _CHUNK
                    v0 = plsc.load_gather(rv, [rr, zc])
                    v1 = plsc.load_gather(rv, [rr, oc])
                    acc0 = w * v0 if acc0 is None else acc0 + w * v0
                    acc1 = w * v1 if acc1 is None else acc1 + w * v1
                prow = q + iot
                plsc.store_scatter(ov, [prow, col0], acc0)
                plsc.store_scatter(ov, [prow, col1], acc1)

        pltpu.sync_copy(ov, out_hbm.at[pl.ds(base + co, _CHUNK), :])


def kernel(x, tables):
    xt = x.T
    tab = tables.reshape(_L * _TSIZE, _F)
    return _ngp_sc(xt, tab)


# speedup vs baseline: 12.6937x; 12.6937x over previous
import dataclasses
import functools

import numpy as np
import jax
import jax.numpy as jnp
from jax import lax
from jax.experimental import pallas as pl
from jax.experimental.pallas import tpu as pltpu
from jax.experimental.pallas import tpu_sc as plsc

_L = 16
_T = 19
_TSIZE = 1 << _T
_F = 2
_B = 262144
_NC, _NS, _LANES = 2, 16, 16
_NW = _NC * _NS
_CHUNK = 1024
_SB = 32
_ROWS = _L * _TSIZE // 8

_bfac = np.exp((np.log(512.0) - np.log(16.0)) / (_L - 1))
_RES = [int(v) for v in np.floor(16.0 * _bfac ** np.arange(_L)).astype(np.int64)]
_PRIME = -1640531535
_MASK = _TSIZE - 1


def _build(b_total, chunk, interpret=False):
    pts = b_total // _NW

    mesh = plsc.VectorSubcoreMesh(core_axis_name="c", subcore_axis_name="s")

    cparams = pltpu.CompilerParams()
    if "needs_layout_passes" in pltpu.CompilerParams.__dataclass_fields__:
        cparams = dataclasses.replace(cparams, needs_layout_passes=False)
    if "use_tc_tiling_on_sc" in pltpu.CompilerParams.__dataclass_fields__:
        cparams = dataclasses.replace(cparams, use_tc_tiling_on_sc=False)

    @functools.partial(
        pl.kernel,
        out_type=jax.ShapeDtypeStruct((b_total, _L * _F), jnp.float32),
        mesh=mesh,
        compiler_params=cparams,
        interpret=interpret,
        scratch_types=[
            pltpu.VMEM((chunk, 2), jnp.float32),
            pltpu.VMEM((2, chunk), jnp.float32),
            pltpu.VMEM((chunk, _L * _F), jnp.float32),
            pltpu.VMEM((2, _SB), jnp.float32),
            pltpu.VMEM((4 * _SB,), jnp.int32),
            pltpu.VMEM((4 * _SB,), jnp.int32),
            pltpu.VMEM((4 * _SB, 16), jnp.float32),
            pltpu.SemaphoreType.DMA,
        ],
    )
    def _ngp_sc(x_hbm, tab_hbm, out_hbm, xs, xv, ov, fr, ixv, lov, rv, sem):
        wid = lax.axis_index("c") * jnp.int32(_NS) + lax.axis_index("s")
        base = wid * jnp.int32(pts)

        iot = lax.iota(jnp.int32, _LANES)
        zc = jnp.zeros((_LANES,), jnp.int32)
        oc = jnp.ones((_LANES,), jnp.int32)

        @pl.loop(0, pts, step=chunk)
        def _chunk(co):
            pltpu.sync_copy(x_hbm.at[pl.ds(base + co, chunk), :], xs)

            @pl.loop(0, chunk, step=_LANES)
            def _norm(p):
                sl = pl.ds(p, _LANES)
                rw = p + iot
                lat = plsc.load_gather(xs, [rw, zc])
                lon = plsc.load_gather(xs, [rw, oc])
                xv[0, sl] = (lat + 90.0) / 180.0
                xv[1, sl] = lon / 360.0

            for l in range(_L):
                rf = float(_RES[l])
                rowbase = jnp.int32(l << 16)
                mask = jnp.int32(_MASK)
                prime = jnp.int32(_PRIME)
                c0 = jnp.full((_LANES,), 2 * l, jnp.int32)
                c1 = jnp.full((_LANES,), 2 * l + 1, jnp.int32)

                @pl.loop(0, chunk, step=_SB)
                def _sub(s):
                    for g in range(_SB // _LANES):
                        q = s + jnp.int32(g * _LANES)
                        sx = xv[0, pl.ds(q, _LANES)] * rf
                        sy = xv[1, pl.ds(q, _LANES)] * rf
                        gx = sx.astype(jnp.int32)
                        gy = sy.astype(jnp.int32)
                        fr[0, pl.ds(g * _LANES, _LANES)] = sx - gx.astype(jnp.float32)
                        fr[1, pl.ds(g * _LANES, _LANES)] = sy - gy.astype(jnp.float32)
                        hy0 = gy * prime
                        hy1 = hy0 + prime
                        gx1 = gx + jnp.int32(1)
                        for c, h in enumerate((
                            (gx ^ hy0) & mask,
                            (gx1 ^ hy0) & mask,
                            (gx ^ hy1) & mask,
                            (gx1 ^ hy1) & mask,
                        )):
                            sl = pl.ds(c * _SB + g * _LANES, _LANES)
                            ixv[sl] = rowbase + (h >> 3)
                            lov[sl] = (h & jnp.int32(7)) << 1

                    pltpu.async_copy(tab_hbm.at[ixv], rv, sem).wait()

                    for g in range(_SB // _LANES):
                        fx = fr[0, pl.ds(g * _LANES, _LANES)]
                        fy = fr[1, pl.ds(g * _LANES, _LANES)]
                        wx1 = fx
                        wx0 = 1.0 - fx
                        wy1 = fy
                        wy0 = 1.0 - fy
                        acc0 = None
                        acc1 = None
                        for c, w in enumerate((wx0 * wy0, wx1 * wy0,
                                               wx0 * wy1, wx1 * wy1)):
                            r = iot + jnp.int32(c * _SB + g * _LANES)
                            ln = lov[pl.ds(c * _SB + g * _LANES, _LANES)]
                            v0 = plsc.load_gather(rv, [r, ln])
                            v1 = plsc.load_gather(rv, [r, ln + oc])
                            acc0 = w * v0 if acc0 is None else acc0 + w * v0
                            acc1 = w * v1 if acc1 is None else acc1 + w * v1
                        prow = s + jnp.int32(g * _LANES) + iot
                        plsc.store_scatter(ov, [prow, c0], acc0)
                        plsc.store_scatter(ov, [prow, c1], acc1)

            pltpu.sync_copy(ov, out_hbm.at[pl.ds(base + co, chunk), :])

    return _ngp_sc


_ngp_sc_cached = None


def kernel(x, tables):
    global _ngp_sc_cached
    x64_was_on = jax.config.jax_enable_x64
    jax.config.update("jax_enable_x64", False)
    try:
        if _ngp_sc_cached is None:
            _ngp_sc_cached = _build(_B, _CHUNK)
        tab = tables.reshape(_ROWS, 8 * _F)
        out = _ngp_sc_cached(x, tab)
    finally:
        jax.config.update("jax_enable_x64", x64_was_on)
    return out


# speedup vs baseline: 15.2555x; 1.2018x over previous
import dataclasses
import functools

import numpy as np
import jax
import jax.numpy as jnp
from jax import lax
from jax.experimental import pallas as pl
from jax.experimental.pallas import tpu as pltpu
from jax.experimental.pallas import tpu_sc as plsc

_L = 16
_T = 19
_TSIZE = 1 << _T
_F = 2
_B = 262144
_NC, _NS, _LANES = 2, 16, 16
_NW = _NC * _NS
_CHUNK = 1024
_SB = 32
_K = 8
_ROWS = _L * _TSIZE // 8

_bfac = np.exp((np.log(512.0) - np.log(16.0)) / (_L - 1))
_RES = [int(v) for v in np.floor(16.0 * _bfac ** np.arange(_L)).astype(np.int64)]
_PRIME = -1640531535
_MASK = _TSIZE - 1


def _build(b_total, chunk, interpret=False):
    pts = b_total // _NW
    nstr = (chunk // _SB) * _L
    ngrp = _SB // _LANES

    mesh = plsc.VectorSubcoreMesh(core_axis_name="c", subcore_axis_name="s")

    cparams = pltpu.CompilerParams()
    if "needs_layout_passes" in pltpu.CompilerParams.__dataclass_fields__:
        cparams = dataclasses.replace(cparams, needs_layout_passes=False)
    if "use_tc_tiling_on_sc" in pltpu.CompilerParams.__dataclass_fields__:
        cparams = dataclasses.replace(cparams, use_tc_tiling_on_sc=False)

    @functools.partial(
        pl.kernel,
        out_type=jax.ShapeDtypeStruct((b_total, _L * _F), jnp.float32),
        mesh=mesh,
        compiler_params=cparams,
        interpret=interpret,
        scratch_types=(
            [
                pltpu.VMEM((chunk, 2), jnp.float32),
                pltpu.VMEM((2, chunk), jnp.float32),
                pltpu.VMEM((chunk, _L * _F), jnp.float32),
                pltpu.VMEM((_L, _LANES), jnp.float32),
            ]
            + [pltpu.VMEM((4 * _SB,), jnp.int32) for _ in range(_K)]
            + [pltpu.VMEM((2, _SB), jnp.float32) for _ in range(_K)]
            + [pltpu.VMEM((4 * _SB,), jnp.int32) for _ in range(_K)]
            + [pltpu.VMEM((4 * _SB, 16), jnp.float32) for _ in range(_K)]
            + [pltpu.SemaphoreType.DMA for _ in range(_K)]
        ),
    )
    def _ngp_sc(x_hbm, resf_hbm, tab_hbm, out_hbm, xs, xv, ov, resf, *rest):
        lovs = rest[0:_K]
        frs = rest[_K:2 * _K]
        ixvs = rest[2 * _K:3 * _K]
        rvs = rest[3 * _K:4 * _K]
        sems = rest[4 * _K:5 * _K]

        wid = lax.axis_index("c") * jnp.int32(_NS) + lax.axis_index("s")
        base = wid * jnp.int32(pts)

        iot = lax.iota(jnp.int32, _LANES)
        zc = jnp.zeros((_LANES,), jnp.int32)
        oc = jnp.ones((_LANES,), jnp.int32)
        mask = jnp.int32(_MASK)
        prime = jnp.int32(_PRIME)

        pltpu.sync_copy(resf_hbm, resf)

        def hash_fire(u, k):
            lvl = u >> 5
            soff = (u & jnp.int32(31)) << 5
            rowbase = lvl << 16
            rfv = resf[lvl, pl.ds(0, _LANES)]
            for g in range(ngrp):
                q = soff + jnp.int32(g * _LANES)
                sx = xv[0, pl.ds(q, _LANES)] * rfv
                sy = xv[1, pl.ds(q, _LANES)] * rfv
                gx = sx.astype(jnp.int32)
                gy = sy.astype(jnp.int32)
                frs[k][0, pl.ds(g * _LANES, _LANES)] = sx - gx.astype(jnp.float32)
                frs[k][1, pl.ds(g * _LANES, _LANES)] = sy - gy.astype(jnp.float32)
                hy0 = gy * prime
                hy1 = hy0 + prime
                gx1 = gx + jnp.int32(1)
                for c, h in enumerate((
                    (gx ^ hy0) & mask,
                    (gx1 ^ hy0) & mask,
                    (gx ^ hy1) & mask,
                    (gx1 ^ hy1) & mask,
                )):
                    sl = pl.ds(c * _SB + g * _LANES, _LANES)
                    ixvs[k][sl] = rowbase + (h >> 3)
                    lovs[k][sl] = (h & jnp.int32(7)) << 1
            pltpu.async_copy(tab_hbm.at[ixvs[k]], rvs[k], sems[k])

        def wait_interp(u, k):
            pltpu.make_async_copy(tab_hbm.at[ixvs[k]], rvs[k], sems[k]).wait()
            lvl = u >> 5
            soff = (u & jnp.int32(31)) << 5
            c0v = zc + (lvl << 1)
            c1v = c0v + oc
            for g in range(ngrp):
                fx = frs[k][0, pl.ds(g * _LANES, _LANES)]
                fy = frs[k][1, pl.ds(g * _LANES, _LANES)]
                wx1 = fx
                wx0 = 1.0 - fx
                wy1 = fy
                wy0 = 1.0 - fy
                acc0 = None
                acc1 = None
                for c, w in enumerate((wx0 * wy0, wx1 * wy0,
                                       wx0 * wy1, wx1 * wy1)):
                    r = iot + jnp.int32(c * _SB + g * _LANES)
                    ln = lovs[k][pl.ds(c * _SB + g * _LANES, _LANES)]
                    v0 = plsc.load_gather(rvs[k], [r, ln])
                    v1 = plsc.load_gather(rvs[k], [r, ln + oc])
                    acc0 = w * v0 if acc0 is None else acc0 + w * v0
                    acc1 = w * v1 if acc1 is None else acc1 + w * v1
                prow = soff + jnp.int32(g * _LANES) + iot
                plsc.store_scatter(ov, [prow, c0v], acc0)
                plsc.store_scatter(ov, [prow, c1v], acc1)

        @pl.loop(0, pts, step=chunk)
        def _chunk(co):
            pltpu.sync_copy(x_hbm.at[pl.ds(base + co, chunk), :], xs)

            @pl.loop(0, chunk, step=_LANES)
            def _norm(p):
                sl = pl.ds(p, _LANES)
                rw = p + iot
                lat = plsc.load_gather(xs, [rw, zc])
                lon = plsc.load_gather(xs, [rw, oc])
                xv[0, sl] = (lat + 90.0) / 180.0
                xv[1, sl] = lon / 360.0

            for k in range(_K):
                hash_fire(jnp.int32(k), k)

            @pl.loop(0, (nstr - _K) // _K)
            def _steady(it):
                u0 = it * jnp.int32(_K)
                for k in range(_K):
                    u = u0 + jnp.int32(k)
                    wait_interp(u, k)
                    hash_fire(u + jnp.int32(_K), k)

            for k in range(_K):
                wait_interp(jnp.int32(nstr - _K + k), k)

            pltpu.sync_copy(ov, out_hbm.at[pl.ds(base + co, chunk), :])

    return _ngp_sc


_ngp_sc_cached = None


def kernel(x, tables):
    global _ngp_sc_cached
    x64_was_on = jax.config.jax_enable_x64
    jax.config.update("jax_enable_x64", False)
    try:
        if _ngp_sc_cached is None:
            _ngp_sc_cached = _build(_B, _CHUNK)
        resf = jnp.tile(jnp.asarray([float(r) for r in _RES], dtype=jnp.float32)[:, None], (1, _LANES))
        tab = tables.reshape(_ROWS, 8 * _F)
        out = _ngp_sc_cached(x, resf, tab)
    finally:
        jax.config.update("jax_enable_x64", x64_was_on)
    return out


# speedup vs baseline: 15.2735x; 1.0012x over previous
import dataclasses
import functools

import numpy as np
import jax
import jax.numpy as jnp
from jax import lax
from jax.experimental import pallas as pl
from jax.experimental.pallas import tpu as pltpu
from jax.experimental.pallas import tpu_sc as plsc

_L = 16
_T = 19
_TSIZE = 1 << _T
_F = 2
_B = 262144
_NC, _NS, _LANES = 2, 16, 16
_NW = _NC * _NS
_CHUNK = 1024
_SB = 32
_K = 8
_ROWS = _L * _TSIZE // 8

_bfac = np.exp((np.log(512.0) - np.log(16.0)) / (_L - 1))
_RES = [int(v) for v in np.floor(16.0 * _bfac ** np.arange(_L)).astype(np.int64)]
_PRIME = -1640531535
_MASK = _TSIZE - 1


def _build(b_total, chunk, interpret=False):
    pts = b_total // _NW
    nstr = (chunk // _SB) * _L
    ngrp = _SB // _LANES

    mesh = plsc.VectorSubcoreMesh(core_axis_name="c", subcore_axis_name="s")

    cparams = pltpu.CompilerParams()
    if "needs_layout_passes" in pltpu.CompilerParams.__dataclass_fields__:
        cparams = dataclasses.replace(cparams, needs_layout_passes=False)
    if "use_tc_tiling_on_sc" in pltpu.CompilerParams.__dataclass_fields__:
        cparams = dataclasses.replace(cparams, use_tc_tiling_on_sc=False)

    @functools.partial(
        pl.kernel,
        out_type=jax.ShapeDtypeStruct((b_total, _L * _F), jnp.float32),
        mesh=mesh,
        compiler_params=cparams,
        interpret=interpret,
        scratch_types=(
            [
                pltpu.VMEM((chunk, 2), jnp.float32),
                pltpu.VMEM((2, chunk), jnp.float32),
                pltpu.VMEM((chunk, _L * _F), jnp.float32),
                pltpu.VMEM((_L, _LANES), jnp.float32),
            ]
            + [pltpu.VMEM((4 * _SB,), jnp.int32) for _ in range(_K)]
            + [pltpu.VMEM((2, _SB), jnp.float32) for _ in range(_K)]
            + [pltpu.VMEM((4 * _SB,), jnp.int32) for _ in range(_K)]
            + [pltpu.VMEM((4 * _SB, 16), jnp.float32) for _ in range(_K)]
            + [pltpu.SemaphoreType.DMA for _ in range(_K)]
        ),
    )
    def _ngp_sc(x_hbm, resf_hbm, tab_hbm, out_hbm, xs, xv, ov, resf, *rest):
        lovs = rest[0:_K]
        frs = rest[_K:2 * _K]
        ixvs = rest[2 * _K:3 * _K]
        rvs = rest[3 * _K:4 * _K]
        sems = rest[4 * _K:5 * _K]

        wid = lax.axis_index("c") * jnp.int32(_NS) + lax.axis_index("s")
        base = wid * jnp.int32(pts)

        iot = lax.iota(jnp.int32, _LANES)
        zc = jnp.zeros((_LANES,), jnp.int32)
        oc = jnp.ones((_LANES,), jnp.int32)
        mask = jnp.int32(_MASK)
        prime = jnp.int32(_PRIME)

        pltpu.sync_copy(resf_hbm, resf)

        def hash_fire(u, k):
            lvl = u >> 5
            soff = (u & jnp.int32(31)) << 5
            rowbase = lvl << 16
            rfv = resf[lvl, pl.ds(0, _LANES)]
            for g in range(ngrp):
                q = soff + jnp.int32(g * _LANES)
                sx = xv[0, pl.ds(q, _LANES)] * rfv
                sy = xv[1, pl.ds(q, _LANES)] * rfv
                gx = sx.astype(jnp.int32)
                gy = sy.astype(jnp.int32)
                frs[k][0, pl.ds(g * _LANES, _LANES)] = sx - gx.astype(jnp.float32)
                frs[k][1, pl.ds(g * _LANES, _LANES)] = sy - gy.astype(jnp.float32)
                hy0 = gy * prime
                hy1 = hy0 + prime
                gx1 = gx + jnp.int32(1)
                for c, h in enumerate((
                    (gx ^ hy0) & mask,
                    (gx1 ^ hy0) & mask,
                    (gx ^ hy1) & mask,
                    (gx1 ^ hy1) & mask,
                )):
                    sl = pl.ds(c * _SB + g * _LANES, _LANES)
                    ixvs[k][sl] = rowbase + (h >> 3)
                    lovs[k][sl] = (h & jnp.int32(7)) << 1
            pltpu.async_copy(tab_hbm.at[ixvs[k]], rvs[k], sems[k])

        def wait_interp(u, k):
            pltpu.make_async_copy(tab_hbm.at[ixvs[k]], rvs[k], sems[k]).wait()
            lvl = u >> 5
            soff = (u & jnp.int32(31)) << 5
            c0v = zc + (lvl << 1)
            c1v = c0v + oc
            for g in range(ngrp):
                fx = frs[k][0, pl.ds(g * _LANES, _LANES)]
                fy = frs[k][1, pl.ds(g * _LANES, _LANES)]
                wx1 = fx
                wx0 = 1.0 - fx
                wy1 = fy
                wy0 = 1.0 - fy
                acc0 = None
                acc1 = None
                for c, w in enumerate((wx0 * wy0, wx1 * wy0,
                                       wx0 * wy1, wx1 * wy1)):
                    r = iot + jnp.int32(c * _SB + g * _LANES)
                    ln = lovs[k][pl.ds(c * _SB + g * _LANES, _LANES)]
                    v0 = plsc.load_gather(rvs[k], [r, ln])
                    v1 = plsc.load_gather(rvs[k], [r, ln + oc])
                    acc0 = w * v0 if acc0 is None else acc0 + w * v0
                    acc1 = w * v1 if acc1 is None else acc1 + w * v1
                prow = soff + jnp.int32(g * _LANES) + iot
                plsc.store_scatter(ov, [prow, c0v], acc0)
                plsc.store_scatter(ov, [prow, c1v], acc1)

        @pl.loop(0, pts, step=chunk)
        def _chunk(co):
            pltpu.sync_copy(x_hbm.at[pl.ds(base + co, chunk), :], xs)

            @pl.loop(0, chunk, step=_LANES)
            def _norm(p):
                sl = pl.ds(p, _LANES)
                rw = p + iot
                lat = plsc.load_gather(xs, [rw, zc])
                lon = plsc.load_gather(xs, [rw, oc])
                xv[0, sl] = (lat + 90.0) / 180.0
                xv[1, sl] = lon / 360.0

            for k in range(_K):
                hash_fire(jnp.int32(k), k)

            @pl.loop(0, (nstr - _K) // _K)
            def _steady(it):
                u0 = it * jnp.int32(_K)
                for k in range(_K):
                    u = u0 + jnp.int32(k)
                    wait_interp(u, k)
                    hash_fire(u + jnp.int32(_K), k)

            for k in range(_K):
                wait_interp(jnp.int32(nstr - _K + k), k)

            pltpu.sync_copy(ov, out_hbm.at[pl.ds(base + co, chunk), :])

    return _ngp_sc


_ngp_sc_cached = None


def kernel(x, tables):
    global _ngp_sc_cached
    x64_was_on = jax.config.jax_enable_x64
    jax.config.update("jax_enable_x64", False)
    try:
        if _ngp_sc_cached is None:
            _ngp_sc_cached = _build(_B, _CHUNK)
        resf = jnp.tile(jnp.asarray([float(r) for r in _RES], dtype=jnp.float32)[:, None], (1, _LANES))
        tab = tables.reshape(_ROWS, 8 * _F) + 0.0
        out = _ngp_sc_cached(x, resf, tab)
    finally:
        jax.config.update("jax_enable_x64", x64_was_on)
    return out


# speedup vs baseline: 76.7678x; 5.0262x over previous
import dataclasses
import functools

import numpy as np
import jax
import jax.numpy as jnp
from jax import lax
from jax.experimental import pallas as pl
from jax.experimental.pallas import tpu as pltpu
from jax.experimental.pallas import tpu_sc as plsc

_L = 16
_T = 19
_TSIZE = 1 << _T
_F = 2
_B = 262144
_NC, _NS, _LANES = 2, 16, 16
_NW = _NC * _NS
_CHUNK = 1024
_SB = 16
_K = 8
_ROWS = _L * _TSIZE // 8

_bfac = np.exp((np.log(512.0) - np.log(16.0)) / (_L - 1))
_RES = [int(v) for v in np.floor(16.0 * _bfac ** np.arange(_L)).astype(np.int64)]
_PRIME = -1640531535
_MASK = _TSIZE - 1


def _build(b_total, chunk, interpret=False):
    pts = b_total // _NW
    nsb = chunk // _SB
    nstr = nsb * _L
    ngrp = _SB // _LANES

    mesh = plsc.VectorSubcoreMesh(core_axis_name="c", subcore_axis_name="s")

    cparams = pltpu.CompilerParams()
    if "needs_layout_passes" in pltpu.CompilerParams.__dataclass_fields__:
        cparams = dataclasses.replace(cparams, needs_layout_passes=False)
    if "use_tc_tiling_on_sc" in pltpu.CompilerParams.__dataclass_fields__:
        cparams = dataclasses.replace(cparams, use_tc_tiling_on_sc=False)

    @functools.partial(
        pl.kernel,
        out_type=jax.ShapeDtypeStruct((b_total, _L * _F), jnp.float32),
        mesh=mesh,
        compiler_params=cparams,
        interpret=interpret,
        scratch_types=(
            [
                pltpu.VMEM((chunk, 2), jnp.float32),
                pltpu.VMEM((2, chunk), jnp.float32),
                pltpu.VMEM((chunk, _L * _F), jnp.float32),
                pltpu.VMEM((_L, _LANES), jnp.float32),
            ]
            + [pltpu.VMEM((4 * _SB,), jnp.int32) for _ in range(_K)]
            + [pltpu.VMEM((2, _SB), jnp.float32) for _ in range(_K)]
            + [pltpu.VMEM((8 * _SB,), jnp.int32) for _ in range(_K)]
            + [pltpu.VMEM((8 * _SB, 16), jnp.float32) for _ in range(_K)]
            + [pltpu.SemaphoreType.DMA for _ in range(_K)]
        ),
    )
    def _ngp_sc(x_hbm, resf_hbm, tab_hbm, out_hbm, xs, xv, ov, resf, *rest):
        lovs = rest[0:_K]
        frs = rest[_K:2 * _K]
        ixvs = rest[2 * _K:3 * _K]
        rvs = rest[3 * _K:4 * _K]
        sems = rest[4 * _K:5 * _K]

        wid = lax.axis_index("c") * jnp.int32(_NS) + lax.axis_index("s")
        base = wid * jnp.int32(pts)

        iot = lax.iota(jnp.int32, _LANES)
        zc = jnp.zeros((_LANES,), jnp.int32)
        oc = jnp.ones((_LANES,), jnp.int32)
        mask = jnp.int32(_MASK)
        prime = jnp.int32(_PRIME)

        pltpu.sync_copy(resf_hbm, resf)

        def hash_fire(u, k):
            lvl = u // jnp.int32(nsb)
            soff = (u % jnp.int32(nsb)) * jnp.int32(_SB)
            rowbase = lvl << 16
            rfv = resf[lvl, pl.ds(0, _LANES)]
            for g in range(ngrp):
                q = soff + jnp.int32(g * _LANES)
                sx = xv[0, pl.ds(q, _LANES)] * rfv
                sy = xv[1, pl.ds(q, _LANES)] * rfv
                gx = sx.astype(jnp.int32)
                gy = sy.astype(jnp.int32)
                frs[k][0, pl.ds(g * _LANES, _LANES)] = sx - gx.astype(jnp.float32)
                frs[k][1, pl.ds(g * _LANES, _LANES)] = sy - gy.astype(jnp.float32)
                hy0 = gy * prime
                hy1 = hy0 + prime
                gx1 = gx + jnp.int32(1)
                for c, h in enumerate((
                    (gx ^ hy0) & mask,
                    (gx1 ^ hy0) & mask,
                    (gx ^ hy1) & mask,
                    (gx1 ^ hy1) & mask,
                )):
                    sl = pl.ds(c * _SB + g * _LANES, _LANES)
                    row0 = rowbase + (h >> 4) + ((h >> 7) << 3)
                    ixvs[k][sl] = row0
                    ixvs[k][pl.ds(4 * _SB + c * _SB + g * _LANES, _LANES)] = (
                        row0 + jnp.int32(8))
                    lovs[k][sl] = h & jnp.int32(15)
            pltpu.async_copy(tab_hbm.at[ixvs[k]], rvs[k], sems[k])

        def wait_interp(u, k):
            pltpu.make_async_copy(tab_hbm.at[ixvs[k]], rvs[k], sems[k]).wait()
            lvl = u // jnp.int32(nsb)
            soff = (u % jnp.int32(nsb)) * jnp.int32(_SB)
            c0v = zc + (lvl << 1)
            c1v = c0v + oc
            for g in range(ngrp):
                fx = frs[k][0, pl.ds(g * _LANES, _LANES)]
                fy = frs[k][1, pl.ds(g * _LANES, _LANES)]
                wx1 = fx
                wx0 = 1.0 - fx
                wy1 = fy
                wy0 = 1.0 - fy
                acc0 = None
                acc1 = None
                for c, w in enumerate((wx0 * wy0, wx1 * wy0,
                                       wx0 * wy1, wx1 * wy1)):
                    r = iot + jnp.int32(c * _SB + g * _LANES)
                    ln = lovs[k][pl.ds(c * _SB + g * _LANES, _LANES)]
                    v0 = plsc.load_gather(rvs[k], [r, ln])
                    v1 = plsc.load_gather(rvs[k], [r + jnp.int32(4 * _SB), ln])
                    acc0 = w * v0 if acc0 is None else acc0 + w * v0
                    acc1 = w * v1 if acc1 is None else acc1 + w * v1
                prow = soff + jnp.int32(g * _LANES) + iot
                plsc.store_scatter(ov, [prow, c0v], acc0)
                plsc.store_scatter(ov, [prow, c1v], acc1)

        @pl.loop(0, pts, step=chunk)
        def _chunk(co):
            pltpu.sync_copy(x_hbm.at[pl.ds(base + co, chunk), :], xs)

            @pl.loop(0, chunk, step=_LANES)
            def _norm(p):
                sl = pl.ds(p, _LANES)
                rw = p + iot
                lat = plsc.load_gather(xs, [rw, zc])
                lon = plsc.load_gather(xs, [rw, oc])
                xv[0, sl] = (lat + 90.0) / 180.0
                xv[1, sl] = lon / 360.0

            for k in range(_K):
                hash_fire(jnp.int32(k), k)

            @pl.loop(0, (nstr - _K) // _K)
            def _steady(it):
                u0 = it * jnp.int32(_K)
                for k in range(_K):
                    u = u0 + jnp.int32(k)
                    wait_interp(u, k)
                    hash_fire(u + jnp.int32(_K), k)

            for k in range(_K):
                wait_interp(jnp.int32(nstr - _K + k), k)

            pltpu.sync_copy(ov, out_hbm.at[pl.ds(base + co, chunk), :])

    return _ngp_sc


_ngp_sc_cached = None


def kernel(x, tables):
    global _ngp_sc_cached
    x64_was_on = jax.config.jax_enable_x64
    jax.config.update("jax_enable_x64", False)
    try:
        if _ngp_sc_cached is None:
            _ngp_sc_cached = _build(_B, _CHUNK)
        resf = jnp.tile(jnp.asarray([float(r) for r in _RES], dtype=jnp.float32)[:, None], (1, _LANES))
        tab = (tables.reshape(_L, _TSIZE // 128, 128, _F)
               .transpose(0, 1, 3, 2)
               .reshape(_ROWS, 16))
        out = _ngp_sc_cached(x, resf, tab)
    finally:
        jax.config.update("jax_enable_x64", x64_was_on)
    return out
